# Initial kernel scaffold; baseline (speedup 1.0000x reference)
import jax
import jax.numpy as jnp
from jax.experimental import pallas as pl


def kernel(inputs, leaf_logits, w1, w2, w3, w4, w5, w6, wr):
    raise NotImplementedError("write your pallas kernel here")


# speedup vs baseline: 662.1063x; 662.1063x over previous
import jax
import jax.numpy as jnp
from jax.experimental import pallas as pl
from jax.experimental.pallas import tpu as pltpu

_NUM_VARS = 64
_K = 32
_V = 256
_B = 512


def _circuit_body(inp_ref, leaf_ref, w1_ref, w2_ref, w3_ref, w4_ref,
                  w5_ref, w6_ref, wr_ref, out_ref):
    iota_vb = jax.lax.broadcasted_iota(jnp.int32, (_V, _B), 0)
    xs = []
    for v in range(_NUM_VARS):
        leaf_v = leaf_ref[v]
        mlf = jnp.max(leaf_v, axis=1, keepdims=True)
        lse = jnp.log(jnp.sum(jnp.exp(leaf_v - mlf), axis=1, keepdims=True)) + mlf
        onehot = (iota_vb == inp_ref[v:v + 1, :]).astype(jnp.bfloat16)
        g = jnp.dot(leaf_v.astype(jnp.bfloat16), onehot,
                    preferred_element_type=jnp.float32)
        xs.append(g - lse)

    for w_ref in (w1_ref, w2_ref, w3_ref, w4_ref, w5_ref, w6_ref):
        r_count = w_ref.shape[0]
        nxt = []
        for r in range(r_count):
            lft = xs[2 * r]
            rgt = xs[2 * r + 1]
            m_l = jnp.max(lft, axis=0, keepdims=True)
            m_r = jnp.max(rgt, axis=0, keepdims=True)
            e_l = jnp.exp(lft - m_l).astype(jnp.bfloat16)
            e_r = jnp.exp(rgt - m_r).astype(jnp.bfloat16)
            prod = (e_l[:, None, :] * e_r[None, :, :]).reshape(_K * _K, _B)
            w_v = w_ref[r]
            m_w = jnp.max(w_v, axis=1, keepdims=True)
            w_e = jnp.exp(w_v - m_w)
            w_p = (w_e / jnp.sum(w_e, axis=1, keepdims=True)).astype(jnp.bfloat16)
            dot = jnp.dot(w_p, prod, preferred_element_type=jnp.float32)
            nxt.append(jnp.log(dot + 1e-37) + (m_l + m_r))
        xs = nxt

    wr_col = wr_ref[...]
    m_w = jnp.max(wr_col)
    lse_w = jnp.log(jnp.sum(jnp.exp(wr_col - m_w))) + m_w
    t = xs[0] + (wr_col - lse_w)
    m_t = jnp.max(t, axis=0, keepdims=True)
    out_ref[...] = jnp.log(jnp.sum(jnp.exp(t - m_t), axis=0, keepdims=True)) + m_t


def kernel(inputs, leaf_logits, w1, w2, w3, w4, w5, w6, wr):
    lls = pl.pallas_call(
        _circuit_body,
        out_shape=jax.ShapeDtypeStruct((1, _B), jnp.float32),
    )(inputs.T, leaf_logits, w1, w2, w3, w4, w5, w6, wr[:, None])
    return lls.reshape(_B, 1)
